# baseline (speedup 1.0000x reference)
import functools

import jax
import jax.numpy as jnp
from jax import lax
from jax.experimental import pallas as pl
from jax.experimental.pallas import tpu as pltpu
from jax.experimental.pallas import tpu_sc as plsc

D = 64
DP = 128
NC = 2
NS = 16
NW = NC * NS
CHUNK = 128
NBUF = 4
NTB = 2

_mesh = plsc.VectorSubcoreMesh(core_axis_name="c", subcore_axis_name="s")


def _make_sc_lookup(B: int, L: int):
    steps = L
    assert B == NW * CHUNK and steps % NBUF == 0 and steps >= 3 * NBUF

    @functools.partial(
        pl.kernel,
        mesh=_mesh,
        compiler_params=pltpu.CompilerParams(use_tc_tiling_on_sc=True,
                                             needs_layout_passes=False),
        out_type=jax.ShapeDtypeStruct((L, D, B), jnp.float32),
        scratch_types=[
            pltpu.VMEM((steps, CHUNK), jnp.int32),
            pltpu.VMEM((8, CHUNK), jnp.int32),
            pltpu.VMEM((NBUF, CHUNK, DP), jnp.float32),
            pltpu.VMEM((NTB, D, CHUNK), jnp.float32),
        ] + [pltpu.SemaphoreType.DMA] * (2 * NBUF + NTB),
    )
    def sc_lookup(ids_hbm, t2d_hbm, w_hbm, out_hbm, ids_v, dix_v, rows_v,
                  tr_v, *sems):
        dsem = sems[0:NBUF]
        rsem = sems[NBUF:2 * NBUF]
        ssem = sems[2 * NBUF:2 * NBUF + NTB]
        wid = lax.axis_index("s") * NC + lax.axis_index("c")
        b0 = wid * CHUNK
        pltpu.sync_copy(ids_hbm.at[:, pl.ds(b0, CHUNK)], ids_v)

        def issue_d(j, slot):
            pltpu.async_copy(t2d_hbm.at[ids_v.at[j]], dix_v.at[slot],
                             dsem[slot])

        def wait_d(slot):
            pltpu.make_async_copy(t2d_hbm.at[pl.ds(0, CHUNK)],
                                  dix_v.at[slot], dsem[slot]).wait()

        def issue_r(slot):
            pltpu.async_copy(w_hbm.at[dix_v.at[slot]], rows_v.at[slot],
                             rsem[slot])

        def wait_r(slot):
            pltpu.make_async_copy(w_hbm.at[pl.ds(0, CHUNK)],
                                  rows_v.at[slot], rsem[slot]).wait()

        ridxs = [lax.iota(jnp.int32, 16) + i0 for i0 in range(0, CHUNK, 16)]

        def transpose(rslot, tslot):
            rows = rows_v.at[rslot]
            tr = tr_v.at[tslot]

            def quad(k, carry):
                for dd in range(4):
                    d = k * 4 + dd
                    dvec = jnp.full((16,), d, jnp.int32)
                    for i0 in range(CHUNK // 16):
                        vals = plsc.load_gather(rows, [ridxs[i0], dvec])
                        tr[d, pl.ds(i0 * 16, 16)] = vals
                return carry

            lax.fori_loop(0, D // 4, quad, 0)

        def issue_s(j, tslot):
            pltpu.async_copy(tr_v.at[tslot],
                             out_hbm.at[j, :, pl.ds(b0, CHUNK)],
                             ssem[tslot])

        def wait_s(tslot):
            pltpu.make_async_copy(tr_v.at[tslot],
                                  out_hbm.at[0, :, pl.ds(0, CHUNK)],
                                  ssem[tslot]).wait()

        for k in range(NBUF - 1):
            issue_d(k, k)

        for j in range(NBUF):
            if j >= 1:
                pb = (j - 1) % NBUF
                wait_r(pb)
                issue_d(j + NBUF - 1, pb)
                if j >= NTB + 1:
                    wait_s((j - 1) % NTB)
                transpose(pb, (j - 1) % NTB)
                issue_s(j - 1, (j - 1) % NTB)
            else:
                issue_d(j + NBUF - 1, (j - 1) % NBUF)
            wait_d(j % NBUF)
            issue_r(j % NBUF)

        grp = steps // NBUF

        def group(g, carry):
            j0 = g * NBUF
            for b in range(NBUF):
                j = j0 + b
                pb = (b - 1) % NBUF
                pt = (b - 1) % NTB
                wait_r(pb)
                issue_d(j + NBUF - 1, pb)
                wait_s(pt)
                transpose(pb, pt)
                issue_s(j - 1, pt)
                wait_d(b)
                issue_r(b)
            return carry

        lax.fori_loop(1, grp - 1, group, 0)

        j0 = (grp - 1) * NBUF
        for b in range(NBUF):
            j = j0 + b
            pb = (b - 1) % NBUF
            wait_r(pb)
            if j + NBUF - 1 < steps:
                issue_d(j + NBUF - 1, pb)
            wait_s((j - 1) % NTB)
            transpose(pb, (j - 1) % NTB)
            issue_s(j - 1, (j - 1) % NTB)
            wait_d(b)
            issue_r(b)

        wait_r((steps - 1) % NBUF)
        wait_s((steps - 1) % NTB)
        transpose((steps - 1) % NBUF, (steps - 1) % NTB)
        issue_s(steps - 1, (steps - 1) % NTB)
        for t in range(NTB):
            wait_s(t)

    return sc_lookup


def kernel(input_ids, attention_mask, tok2dict, embed_weight):
    B, L = input_ids.shape
    idsT = input_ids.T
    w128 = jnp.pad(embed_weight, ((0, 0), (0, DP - D)))
    outT = _make_sc_lookup(B, L)(idsT, tok2dict, w128)
    emb = outT.transpose(2, 0, 1)
    return (emb, attention_mask.astype(jnp.int32))


# speedup vs baseline: 1.3793x; 1.3793x over previous
import functools

import jax
import jax.numpy as jnp
from jax import lax
from jax.experimental import pallas as pl
from jax.experimental.pallas import tpu as pltpu
from jax.experimental.pallas import tpu_sc as plsc

D = 64
DP = 128
NC = 2
NS = 16
NW = NC * NS
CHUNK = 128
NBUF = 4
NTB = 2

_mesh = plsc.VectorSubcoreMesh(core_axis_name="c", subcore_axis_name="s")


def _make_sc_lookup(B: int, L: int):
    steps = L
    assert B == NW * CHUNK and steps % NBUF == 0 and steps >= 3 * NBUF

    @functools.partial(
        pl.kernel,
        mesh=_mesh,
        compiler_params=pltpu.CompilerParams(use_tc_tiling_on_sc=True,
                                             needs_layout_passes=False),
        out_type=jax.ShapeDtypeStruct((L, D, B), jnp.float32),
        scratch_types=[
            pltpu.VMEM((steps, CHUNK), jnp.int32),
            pltpu.VMEM((8, CHUNK), jnp.int32),
            pltpu.VMEM((NBUF, CHUNK, DP), jnp.float32),
            pltpu.VMEM((NTB, D, CHUNK), jnp.float32),
        ] + [pltpu.SemaphoreType.DMA] * (2 * NBUF + NTB),
    )
    def sc_lookup(ids_hbm, t2d_hbm, w_hbm, out_hbm, ids_v, dix_v, rows_v,
                  tr_v, *sems):
        dsem = sems[0:NBUF]
        rsem = sems[NBUF:2 * NBUF]
        ssem = sems[2 * NBUF:2 * NBUF + NTB]
        wid = lax.axis_index("s") * NC + lax.axis_index("c")
        b0 = wid * CHUNK
        pltpu.sync_copy(ids_hbm.at[:, pl.ds(b0, CHUNK)], ids_v)

        def issue_d(j, slot):
            pltpu.async_copy(t2d_hbm.at[ids_v.at[j]], dix_v.at[slot],
                             dsem[slot])

        def wait_d(slot):
            pltpu.make_async_copy(t2d_hbm.at[pl.ds(0, CHUNK)],
                                  dix_v.at[slot], dsem[slot]).wait()

        def issue_r(slot):
            pltpu.async_copy(w_hbm.at[dix_v.at[slot]], rows_v.at[slot],
                             rsem[slot])

        def wait_r(slot):
            pltpu.make_async_copy(w_hbm.at[pl.ds(0, CHUNK)],
                                  rows_v.at[slot], rsem[slot]).wait()

        ridxs = [lax.iota(jnp.int32, 16) + i0 for i0 in range(0, CHUNK, 16)]

        def transpose(rslot, tslot):
            rows = rows_v.at[rslot]
            tr = tr_v.at[tslot]

            @plsc.parallel_loop(0, D, step=1, unroll=4)
            def body(d):
                dvec = jnp.full((16,), d, jnp.int32)
                for i0 in range(CHUNK // 16):
                    vals = plsc.load_gather(rows, [ridxs[i0], dvec])
                    tr[d, pl.ds(i0 * 16, 16)] = vals

        def issue_s(j, tslot):
            pltpu.async_copy(tr_v.at[tslot],
                             out_hbm.at[j, :, pl.ds(b0, CHUNK)],
                             ssem[tslot])

        def wait_s(tslot):
            pltpu.make_async_copy(tr_v.at[tslot],
                                  out_hbm.at[0, :, pl.ds(0, CHUNK)],
                                  ssem[tslot]).wait()

        for k in range(NBUF - 1):
            issue_d(k, k)

        for j in range(NBUF):
            if j >= 1:
                pb = (j - 1) % NBUF
                wait_r(pb)
                issue_d(j + NBUF - 1, pb)
                if j >= NTB + 1:
                    wait_s((j - 1) % NTB)
                transpose(pb, (j - 1) % NTB)
                issue_s(j - 1, (j - 1) % NTB)
            else:
                issue_d(j + NBUF - 1, (j - 1) % NBUF)
            wait_d(j % NBUF)
            issue_r(j % NBUF)

        grp = steps // NBUF

        def group(g, carry):
            j0 = g * NBUF
            for b in range(NBUF):
                j = j0 + b
                pb = (b - 1) % NBUF
                pt = (b - 1) % NTB
                wait_r(pb)
                issue_d(j + NBUF - 1, pb)
                wait_s(pt)
                transpose(pb, pt)
                issue_s(j - 1, pt)
                wait_d(b)
                issue_r(b)
            return carry

        lax.fori_loop(1, grp - 1, group, 0)

        j0 = (grp - 1) * NBUF
        for b in range(NBUF):
            j = j0 + b
            pb = (b - 1) % NBUF
            wait_r(pb)
            if j + NBUF - 1 < steps:
                issue_d(j + NBUF - 1, pb)
            wait_s((j - 1) % NTB)
            transpose(pb, (j - 1) % NTB)
            issue_s(j - 1, (j - 1) % NTB)
            wait_d(b)
            issue_r(b)

        wait_r((steps - 1) % NBUF)
        wait_s((steps - 1) % NTB)
        transpose((steps - 1) % NBUF, (steps - 1) % NTB)
        issue_s(steps - 1, (steps - 1) % NTB)
        for t in range(NTB):
            wait_s(t)

    return sc_lookup


def kernel(input_ids, attention_mask, tok2dict, embed_weight):
    B, L = input_ids.shape
    idsT = input_ids.T
    w128 = jnp.pad(embed_weight, ((0, 0), (0, DP - D)))
    outT = _make_sc_lookup(B, L)(idsT, tok2dict, w128)
    emb = outT.transpose(2, 0, 1)
    return (emb, attention_mask.astype(jnp.int32))


# speedup vs baseline: 2.1151x; 1.5335x over previous
import functools

import jax
import jax.numpy as jnp
from jax import lax
from jax.experimental import pallas as pl
from jax.experimental.pallas import tpu as pltpu
from jax.experimental.pallas import tpu_sc as plsc

D = 64
DP = 128
NC = 2
NS = 16
NW = NC * NS
CHUNK = 128
NBUF = 4
NTB = 2

_mesh = plsc.VectorSubcoreMesh(core_axis_name="c", subcore_axis_name="s")


def _make_sc_lookup(B: int, L: int):
    steps = L
    assert B == NW * CHUNK and steps % NBUF == 0 and steps >= 3 * NBUF

    @functools.partial(
        pl.kernel,
        mesh=_mesh,
        compiler_params=pltpu.CompilerParams(use_tc_tiling_on_sc=True,
                                             needs_layout_passes=False),
        out_type=jax.ShapeDtypeStruct((L, D, B), jnp.float32),
        scratch_types=[
            pltpu.VMEM((steps, CHUNK), jnp.int32),
            pltpu.VMEM((8, CHUNK), jnp.int32),
            pltpu.VMEM((NBUF, CHUNK, DP), jnp.float32),
            pltpu.VMEM((NTB, D, CHUNK), jnp.float32),
        ] + [pltpu.SemaphoreType.DMA] * (2 * NBUF + NTB),
    )
    def sc_lookup(ids_hbm, t2d_hbm, w_hbm, out_hbm, ids_v, dix_v, rows_v,
                  tr_v, *sems):
        dsem = sems[0:NBUF]
        rsem = sems[NBUF:2 * NBUF]
        ssem = sems[2 * NBUF:2 * NBUF + NTB]
        wid = lax.axis_index("s") * NC + lax.axis_index("c")
        b0 = wid * CHUNK
        pltpu.sync_copy(ids_hbm.at[:, pl.ds(b0, CHUNK)], ids_v)

        def issue_d(j, slot):
            pltpu.async_copy(t2d_hbm.at[ids_v.at[j]], dix_v.at[slot],
                             dsem[slot])

        def wait_d(slot):
            pltpu.make_async_copy(t2d_hbm.at[pl.ds(0, CHUNK)],
                                  dix_v.at[slot], dsem[slot]).wait()

        def issue_r(slot):
            pltpu.async_copy(w_hbm.at[dix_v.at[slot]], rows_v.at[slot],
                             rsem[slot])

        def wait_r(slot):
            pltpu.make_async_copy(w_hbm.at[pl.ds(0, CHUNK)],
                                  rows_v.at[slot], rsem[slot]).wait()

        iota16 = lax.iota(jnp.int32, 16)
        rot = [(iota16 + o) & 15 for o in range(16)]

        def transpose(rslot, tslot):
            rows = rows_v.at[rslot]
            tr = tr_v.at[tslot]

            @plsc.parallel_loop(0, (D // 16) * (CHUNK // 16), step=1,
                                unroll=2)
            def blk(k):
                d0 = (k // (CHUNK // 16)) * 16
                i0 = (k % (CHUNK // 16)) * 16
                ridx = iota16 + i0
                for o in range(16):
                    cidx = rot[o] + d0
                    vals = plsc.load_gather(rows, [ridx, cidx])
                    plsc.store_scatter(tr, [cidx, ridx], vals)

        def issue_s(j, tslot):
            pltpu.async_copy(tr_v.at[tslot],
                             out_hbm.at[j, :, pl.ds(b0, CHUNK)],
                             ssem[tslot])

        def wait_s(tslot):
            pltpu.make_async_copy(tr_v.at[tslot],
                                  out_hbm.at[0, :, pl.ds(0, CHUNK)],
                                  ssem[tslot]).wait()

        for k in range(NBUF - 1):
            issue_d(k, k)

        for j in range(NBUF):
            if j >= 1:
                pb = (j - 1) % NBUF
                wait_r(pb)
                issue_d(j + NBUF - 1, pb)
                if j >= NTB + 1:
                    wait_s((j - 1) % NTB)
                transpose(pb, (j - 1) % NTB)
                issue_s(j - 1, (j - 1) % NTB)
            else:
                issue_d(j + NBUF - 1, (j - 1) % NBUF)
            wait_d(j % NBUF)
            issue_r(j % NBUF)

        grp = steps // NBUF

        def group(g, carry):
            j0 = g * NBUF
            for b in range(NBUF):
                j = j0 + b
                pb = (b - 1) % NBUF
                pt = (b - 1) % NTB
                wait_r(pb)
                issue_d(j + NBUF - 1, pb)
                wait_s(pt)
                transpose(pb, pt)
                issue_s(j - 1, pt)
                wait_d(b)
                issue_r(b)
            return carry

        lax.fori_loop(1, grp - 1, group, 0)

        j0 = (grp - 1) * NBUF
        for b in range(NBUF):
            j = j0 + b
            pb = (b - 1) % NBUF
            wait_r(pb)
            if j + NBUF - 1 < steps:
                issue_d(j + NBUF - 1, pb)
            wait_s((j - 1) % NTB)
            transpose(pb, (j - 1) % NTB)
            issue_s(j - 1, (j - 1) % NTB)
            wait_d(b)
            issue_r(b)

        wait_r((steps - 1) % NBUF)
        wait_s((steps - 1) % NTB)
        transpose((steps - 1) % NBUF, (steps - 1) % NTB)
        issue_s(steps - 1, (steps - 1) % NTB)
        for t in range(NTB):
            wait_s(t)

    return sc_lookup


def kernel(input_ids, attention_mask, tok2dict, embed_weight):
    B, L = input_ids.shape
    idsT = input_ids.T
    w128 = jnp.pad(embed_weight, ((0, 0), (0, DP - D)))
    outT = _make_sc_lookup(B, L)(idsT, tok2dict, w128)
    emb = outT.transpose(2, 0, 1)
    return (emb, attention_mask.astype(jnp.int32))


# speedup vs baseline: 2.5836x; 1.2215x over previous
import functools

import jax
import jax.numpy as jnp
from jax import lax
from jax.experimental import pallas as pl
from jax.experimental.pallas import tpu as pltpu
from jax.experimental.pallas import tpu_sc as plsc

D = 64
DP = 128
NC = 2
NS = 16
NW = NC * NS
CHUNK = 128
NBUF = 4
NTB = 2

_mesh = plsc.VectorSubcoreMesh(core_axis_name="c", subcore_axis_name="s")


def _make_sc_lookup(B: int, L: int):
    steps = L
    assert B == NW * CHUNK and steps % NBUF == 0 and steps >= 3 * NBUF

    @functools.partial(
        pl.kernel,
        mesh=_mesh,
        compiler_params=pltpu.CompilerParams(use_tc_tiling_on_sc=True,
                                             needs_layout_passes=False),
        out_type=jax.ShapeDtypeStruct((L, D, B), jnp.float32),
        scratch_types=[
            pltpu.VMEM((steps, CHUNK), jnp.int32),
            pltpu.VMEM((8, CHUNK), jnp.int32),
            pltpu.VMEM((NBUF, CHUNK, DP), jnp.float32),
            pltpu.VMEM((NTB, D, CHUNK), jnp.float32),
        ] + [pltpu.SemaphoreType.DMA] * (2 * NBUF + NTB),
    )
    def sc_lookup(ids_hbm, t2d_hbm, w_hbm, out_hbm, ids_v, dix_v, rows_v,
                  tr_v, *sems):
        dsem = sems[0:NBUF]
        rsem = sems[NBUF:2 * NBUF]
        ssem = sems[2 * NBUF:2 * NBUF + NTB]
        wid = lax.axis_index("s") * NC + lax.axis_index("c")
        b0 = wid * CHUNK
        pltpu.sync_copy(ids_hbm.at[:, pl.ds(b0, CHUNK)], ids_v)

        def issue_d(j, slot):
            pltpu.async_copy(t2d_hbm.at[ids_v.at[j]], dix_v.at[slot],
                             dsem[slot])

        def wait_d(slot):
            pltpu.make_async_copy(t2d_hbm.at[pl.ds(0, CHUNK)],
                                  dix_v.at[slot], dsem[slot]).wait()

        def issue_r(slot):
            pltpu.async_copy(w_hbm.at[dix_v.at[slot]], rows_v.at[slot],
                             rsem[slot])

        def wait_r(slot):
            pltpu.make_async_copy(w_hbm.at[pl.ds(0, CHUNK)],
                                  rows_v.at[slot], rsem[slot]).wait()

        iota16 = lax.iota(jnp.int32, 16)
        rot = [(iota16 + o) & 15 for o in range(16)]

        def transpose(rslot, tslot):
            rows = rows_v.at[rslot]
            tr = tr_v.at[tslot]

            @plsc.parallel_loop(0, (D // 16) * (CHUNK // 16), step=1,
                                unroll=2)
            def blk(k):
                d0 = (k // (CHUNK // 16)) * 16
                i0 = (k % (CHUNK // 16)) * 16
                ridx = iota16 + i0
                for o in range(16):
                    cidx = rot[o] + d0
                    vals = plsc.load_gather(rows, [ridx, cidx])
                    plsc.store_scatter(tr, [cidx, ridx], vals)

        def issue_s(j, tslot):
            pltpu.async_copy(tr_v.at[tslot],
                             out_hbm.at[j, :, pl.ds(b0, CHUNK)],
                             ssem[tslot])

        def wait_s(tslot):
            pltpu.make_async_copy(tr_v.at[tslot],
                                  out_hbm.at[0, :, pl.ds(0, CHUNK)],
                                  ssem[tslot]).wait()

        def visit(j, b, first_s, last_d):
            pb2 = (b - 2) % NBUF
            t = b % NTB
            wait_r(pb2)
            if not last_d:
                issue_d(j + 2, (b + 2) % NBUF)
            if not first_s:
                wait_s(t)
            transpose(pb2, t)
            issue_s(j - 2, t)
            wait_d(b)
            issue_r(b)

        issue_d(0, 0)
        issue_d(1, 1)
        wait_d(0)
        issue_r(0)
        issue_d(2, 2)
        wait_d(1)
        issue_r(1)
        issue_d(3, 3)

        for j in range(2, 6):
            visit(j, j % NBUF, first_s=(j < 4), last_d=False)

        ngrp = (steps - 2 - 6) // NBUF

        def group(g, carry):
            j0 = 6 + g * NBUF
            for b0 in range(NBUF):
                visit(j0 + b0, (6 + b0) % NBUF, first_s=False, last_d=False)
            return carry

        lax.fori_loop(0, ngrp, group, 0)

        for j in range(steps - 2, steps):
            visit(j, j % NBUF, first_s=False, last_d=True)

        for j in range(steps, steps + 2):
            pb2 = (j - 2) % NBUF
            t = j % NTB
            wait_r(pb2)
            wait_s(t)
            transpose(pb2, t)
            issue_s(j - 2, t)
        for t in range(NTB):
            wait_s(t)

    return sc_lookup


def kernel(input_ids, attention_mask, tok2dict, embed_weight):
    B, L = input_ids.shape
    idsT = input_ids.T
    w128 = jnp.pad(embed_weight, ((0, 0), (0, DP - D)))
    outT = _make_sc_lookup(B, L)(idsT, tok2dict, w128)
    emb = outT.transpose(2, 0, 1)
    return (emb, attention_mask.astype(jnp.int32))


# speedup vs baseline: 2.6204x; 1.0142x over previous
import functools

import jax
import jax.numpy as jnp
from jax import lax
from jax.experimental import pallas as pl
from jax.experimental.pallas import tpu as pltpu
from jax.experimental.pallas import tpu_sc as plsc

D = 64
DP = 128
NC = 2
NS = 16
NW = NC * NS
CHUNK = 128
NBUF = 4
NTB = 2

_mesh = plsc.VectorSubcoreMesh(core_axis_name="c", subcore_axis_name="s")


def _make_sc_lookup(B: int, L: int):
    steps = L
    assert B == NW * CHUNK and steps % NBUF == 0 and steps >= 3 * NBUF

    @functools.partial(
        pl.kernel,
        mesh=_mesh,
        compiler_params=pltpu.CompilerParams(use_tc_tiling_on_sc=True,
                                             needs_layout_passes=False),
        out_type=jax.ShapeDtypeStruct((L, D, B), jnp.float32),
        scratch_types=[
            pltpu.VMEM((steps, CHUNK), jnp.int32),
            pltpu.VMEM((8, CHUNK), jnp.int32),
            pltpu.VMEM((NBUF, CHUNK, DP), jnp.float32),
            pltpu.VMEM((NTB, D, CHUNK), jnp.float32),
        ] + [pltpu.SemaphoreType.DMA] * (8 + NBUF + NTB),
    )
    def sc_lookup(ids_hbm, t2d_hbm, w_hbm, out_hbm, ids_v, dix_v, rows_v,
                  tr_v, *sems):
        dsem = sems[0:8]
        rsem = sems[8:8 + NBUF]
        ssem = sems[8 + NBUF:8 + NBUF + NTB]
        wid = lax.axis_index("s") * NC + lax.axis_index("c")
        b0 = wid * CHUNK
        pltpu.sync_copy(ids_hbm.at[:, pl.ds(b0, CHUNK)], ids_v)

        def issue_d(j, slot):
            pltpu.async_copy(t2d_hbm.at[ids_v.at[j]], dix_v.at[slot],
                             dsem[slot])

        def wait_d(slot):
            pltpu.make_async_copy(t2d_hbm.at[pl.ds(0, CHUNK)],
                                  dix_v.at[slot], dsem[slot]).wait()

        def issue_r(slot, dslot):
            pltpu.async_copy(w_hbm.at[dix_v.at[dslot]], rows_v.at[slot],
                             rsem[slot])

        def wait_r(slot):
            pltpu.make_async_copy(w_hbm.at[pl.ds(0, CHUNK)],
                                  rows_v.at[slot], rsem[slot]).wait()

        iota16 = lax.iota(jnp.int32, 16)
        rot = [(iota16 + o) & 15 for o in range(16)]

        def transpose(rslot, tslot):
            rows = rows_v.at[rslot]
            tr = tr_v.at[tslot]

            @plsc.parallel_loop(0, (D // 16) * (CHUNK // 16), step=1,
                                unroll=2)
            def blk(k):
                d0 = (k // (CHUNK // 16)) * 16
                i0 = (k % (CHUNK // 16)) * 16
                ridx = iota16 + i0
                for o in range(16):
                    cidx = rot[o] + d0
                    vals = plsc.load_gather(rows, [ridx, cidx])
                    plsc.store_scatter(tr, [cidx, ridx], vals)

        def issue_s(j, tslot):
            pltpu.async_copy(tr_v.at[tslot],
                             out_hbm.at[j, :, pl.ds(b0, CHUNK)],
                             ssem[tslot])

        def wait_s(tslot):
            pltpu.make_async_copy(tr_v.at[tslot],
                                  out_hbm.at[0, :, pl.ds(0, CHUNK)],
                                  ssem[tslot]).wait()

        def visit(j, bd, first_s, last_d):
            b = bd % NBUF
            pb3 = (bd - 3) % NBUF
            t = (bd - 3) % NTB
            wait_r(pb3)
            if not last_d:
                issue_d(j + 5, (bd + 5) % 8)
            if not first_s:
                wait_s(t)
            transpose(pb3, t)
            issue_s(j - 3, t)
            wait_d(bd)
            issue_r(b, bd)

        for k in range(5):
            issue_d(k, k)
        for j in range(3):
            wait_d(j)
            issue_r(j % NBUF, j)
            issue_d(j + 5, j + 5)

        for j in range(3, 8):
            visit(j, j % 8, first_s=(j < 5), last_d=False)

        ngrp = (steps - 8 - 8) // 8

        def group(g, carry):
            j0 = 8 + g * 8
            for b0 in range(8):
                visit(j0 + b0, b0, first_s=False, last_d=False)
            return carry

        lax.fori_loop(0, ngrp, group, 0)

        for j in range(8 + 8 * ngrp, steps - 5):
            visit(j, j % 8, first_s=False, last_d=False)
        for j in range(steps - 5, steps):
            visit(j, j % 8, first_s=False, last_d=True)

        for j in range(steps, steps + 3):
            pb3 = (j - 3) % NBUF
            t = (j - 3) % NTB
            wait_r(pb3)
            wait_s(t)
            transpose(pb3, t)
            issue_s(j - 3, t)
        for t in range(NTB):
            wait_s(t)

    return sc_lookup


def kernel(input_ids, attention_mask, tok2dict, embed_weight):
    B, L = input_ids.shape
    idsT = input_ids.T
    w128 = jnp.pad(embed_weight, ((0, 0), (0, DP - D)))
    outT = _make_sc_lookup(B, L)(idsT, tok2dict, w128)
    emb = outT.transpose(2, 0, 1)
    return (emb, attention_mask.astype(jnp.int32))
